# Initial kernel scaffold; baseline (speedup 1.0000x reference)
import functools

import jax
import jax.numpy as jnp
from jax import lax
from jax.experimental import pallas as pl
from jax.experimental.pallas import tpu as pltpu
from jax.experimental.pallas import tpu_sc as plsc

N_N = 10000
E_E = 160000
C = 128
H = 8
V = 16
A = 64
R = 64

NC = 2
NS = 16
NW = NC * NS
LANES = 16

E_PER_W = E_E // NW
EK = 200
N_CHUNKS = E_PER_W // EK
NR_PER_T = N_N // NS
E_PER_T = E_E // NS

_SC_MESH = dict(core_axis_name="c", subcore_axis_name="s")


def _tables_body(an_ref, x0_ref, asrc_ref, adst_ref, u_ref, w_ref):
    an = an_ref[...]
    iot = lax.broadcasted_iota(jnp.int32, (1, 128), 1)
    onehot = (an == iot).astype(jnp.float32)
    tsrc = jnp.dot(onehot, asrc_ref[...], preferred_element_type=jnp.float32)
    tdst = jnp.dot(onehot, adst_ref[...], preferred_element_type=jnp.float32)
    x0 = x0_ref[...]
    u_ref[...] = jnp.concatenate([tsrc, x0], axis=1)
    w_ref[...] = jnp.concatenate([tdst, x0], axis=1)


def _build_tables(an2, x0, asrc_pad, adst_pad):
    bn = 1000
    return pl.pallas_call(
        _tables_body,
        grid=(N_N // bn,),
        in_specs=[
            pl.BlockSpec((bn, 1), lambda i: (i, 0)),
            pl.BlockSpec((bn, C), lambda i: (i, 0)),
            pl.BlockSpec((128, C), lambda i: (0, 0)),
            pl.BlockSpec((128, C), lambda i: (0, 0)),
        ],
        out_specs=[
            pl.BlockSpec((bn, 2 * C), lambda i: (i, 0)),
            pl.BlockSpec((bn, 2 * C), lambda i: (i, 0)),
        ],
        out_shape=[
            jax.ShapeDtypeStruct((N_N, 2 * C), jnp.float32),
            jax.ShapeDtypeStruct((N_N, 2 * C), jnp.float32),
        ],
    )(an2, x0, asrc_pad, adst_pad)


def _edge_gather_body(u_hbm, w_hbm, src_hbm, dst_hbm, out_hbm,
                      sidx, didx, rows_u, rows_w, sem1, sem2):
    wid = lax.axis_index("s") * NC + lax.axis_index("c")
    base = wid * E_PER_W

    def chunk(i, _):
        off = base + i * EK
        pltpu.sync_copy(src_hbm.at[pl.ds(off, EK)], sidx)
        pltpu.sync_copy(dst_hbm.at[pl.ds(off, EK)], didx)
        cu = pltpu.async_copy(u_hbm.at[sidx], rows_u, sem1)
        cw = pltpu.async_copy(w_hbm.at[didx], rows_w, sem2)
        cu.wait()
        cw.wait()

        def addrow(k, _):
            for cc in range(2 * C // LANES):
                sl = pl.ds(cc * LANES, LANES)
                rows_u[k, sl] = rows_u[k, sl] + rows_w[k, sl]
            return 0

        lax.fori_loop(0, EK, addrow, 0)
        pltpu.sync_copy(rows_u, out_hbm.at[pl.ds(off, EK)])
        return 0

    lax.fori_loop(0, N_CHUNKS, chunk, 0)


def _edge_gather(u_tab, w_tab, src, dst):
    f = pl.kernel(
        _edge_gather_body,
        out_type=jax.ShapeDtypeStruct((E_E, 2 * C), jnp.float32),
        mesh=plsc.VectorSubcoreMesh(**_SC_MESH),
        scratch_types=[
            pltpu.VMEM((EK,), jnp.int32),
            pltpu.VMEM((EK,), jnp.int32),
            pltpu.VMEM((EK, 2 * C), jnp.float32),
            pltpu.VMEM((EK, 2 * C), jnp.float32),
            pltpu.SemaphoreType.DMA,
            pltpu.SemaphoreType.DMA,
        ],
    )
    return f(u_tab, w_tab, src, dst)


def _edge_math_body(embcat_ref, ed_ref, wr1_ref, wr2_ref, wa1s_ref, wa1r_ref,
                    avflat_ref, sel_ref, wg_ref, wo_ref, exw_ref, gw_ref):
    embcat = embcat_ref[...]
    emb = embcat[:, :C] + ed_ref[...]
    scalar = embcat[:, C:]
    rad = jax.nn.silu(jnp.dot(emb, wr1_ref[...],
                              preferred_element_type=jnp.float32))
    rad = jax.nn.silu(jnp.dot(rad, wr2_ref[...],
                              preferred_element_type=jnp.float32))
    a = jax.nn.silu(
        jnp.dot(scalar, wa1s_ref[...], preferred_element_type=jnp.float32)
        + jnp.dot(rad, wa1r_ref[...], preferred_element_type=jnp.float32))
    aw = a * avflat_ref[...]
    logits16 = jnp.dot(aw, sel_ref[...], preferred_element_type=jnp.float32)
    exw_ref[...] = jnp.exp(logits16)
    gate = jax.nn.sigmoid(jnp.dot(rad, wg_ref[...],
                                  preferred_element_type=jnp.float32))
    gw_ref[...] = gate * wo_ref[...]


def _edge_math(embcat, edge_distance, wr1, wr2, wa1s, wa1r, avflat, sel, wg,
               wo_row):
    be = 2000
    return pl.pallas_call(
        _edge_math_body,
        grid=(E_E // be,),
        in_specs=[
            pl.BlockSpec((be, 2 * C), lambda i: (i, 0)),
            pl.BlockSpec((be, C), lambda i: (i, 0)),
            pl.BlockSpec((C, R), lambda i: (0, 0)),
            pl.BlockSpec((R, R), lambda i: (0, 0)),
            pl.BlockSpec((C, H * A), lambda i: (0, 0)),
            pl.BlockSpec((R, H * A), lambda i: (0, 0)),
            pl.BlockSpec((1, H * A), lambda i: (0, 0)),
            pl.BlockSpec((H * A, 16), lambda i: (0, 0)),
            pl.BlockSpec((R, H * V), lambda i: (0, 0)),
            pl.BlockSpec((1, H * V), lambda i: (0, 0)),
        ],
        out_specs=[
            pl.BlockSpec((be, 16), lambda i: (i, 0)),
            pl.BlockSpec((be, H * V), lambda i: (i, 0)),
        ],
        out_shape=[
            jax.ShapeDtypeStruct((E_E, 16), jnp.float32),
            jax.ShapeDtypeStruct((E_E, H * V), jnp.float32),
        ],
    )(embcat, edge_distance, wr1, wr2, wa1s, wa1r, avflat, sel, wg, wo_row)


def _den_body(exw_hbm, dst_hbm, deng_hbm, zbuf, idxb, rowsb, dshared, sem):
    cid = lax.axis_index("c")
    sid = lax.axis_index("s")
    wid = sid * NC + cid

    def zrow(j, _):
        zbuf[j, :] = jnp.zeros((LANES,), jnp.float32)
        return 0

    lax.fori_loop(0, NR_PER_T, zrow, 0)
    pltpu.sync_copy(zbuf, dshared.at[pl.ds(sid * NR_PER_T, NR_PER_T)])
    plsc.subcore_barrier()

    base = wid * E_PER_W

    def chunk(i, _):
        off = base + i * EK
        pltpu.sync_copy(dst_hbm.at[pl.ds(off, EK)], idxb)
        pltpu.sync_copy(exw_hbm.at[pl.ds(off, EK)], rowsb)
        pltpu.sync_copy(rowsb, dshared.at[idxb], add=True)
        return 0

    lax.fori_loop(0, N_CHUNKS, chunk, 0)
    plsc.subcore_barrier()

    gbase = sid * E_PER_T

    def gchunk(i, _):
        off = gbase + i * EK
        pltpu.sync_copy(dst_hbm.at[pl.ds(off, EK)], idxb)
        pltpu.async_copy(dshared.at[idxb], rowsb, sem).wait()
        pltpu.sync_copy(rowsb, deng_hbm.at[cid, pl.ds(off, EK)])
        return 0

    lax.fori_loop(0, E_PER_T // EK, gchunk, 0)


def _den_scatter_gather(exw, dst):
    f = pl.kernel(
        _den_body,
        out_type=jax.ShapeDtypeStruct((NC, E_E, 16), jnp.float32),
        mesh=plsc.VectorSubcoreMesh(**_SC_MESH),
        scratch_types=[
            pltpu.VMEM((NR_PER_T, 16), jnp.float32),
            pltpu.VMEM((EK,), jnp.int32),
            pltpu.VMEM((EK, 16), jnp.float32),
            pltpu.VMEM_SHARED((N_N, 16), jnp.float32),
            pltpu.SemaphoreType.DMA,
        ],
    )
    return f(exw, dst)


def _u_body(exw_ref, deng_ref, gw_ref, rep_ref, wvt_ref, u_ref):
    ex = exw_ref[...][:, :H]
    den = deng_ref[0][:, :H] + deng_ref[1][:, :H]
    alpha = ex / (den + 1e-9)
    alpha_rep = jnp.dot(alpha, rep_ref[...], preferred_element_type=jnp.float32)
    u_ref[...] = jnp.dot(alpha_rep * gw_ref[...], wvt_ref[...],
                         preferred_element_type=jnp.float32)


def _u_project(exw, deng, gw, rep, wvt):
    be = 2000
    return pl.pallas_call(
        _u_body,
        grid=(E_E // be,),
        in_specs=[
            pl.BlockSpec((be, 16), lambda i: (i, 0)),
            pl.BlockSpec((NC, be, 16), lambda i: (0, i, 0)),
            pl.BlockSpec((be, H * V), lambda i: (i, 0)),
            pl.BlockSpec((H, H * V), lambda i: (0, 0)),
            pl.BlockSpec((H * V, C), lambda i: (0, 0)),
        ],
        out_specs=pl.BlockSpec((be, C), lambda i: (i, 0)),
        out_shape=jax.ShapeDtypeStruct((E_E, C), jnp.float32),
    )(exw, deng, gw, rep, wvt)


def _src_body(x134_hbm, u_hbm, src_hbm, dst_hbm, fsrc_hbm, ssum_hbm,
              sidx, didx, xsb, ub, cb, fshared, sshared, sem):
    cid = lax.axis_index("c")
    sid = lax.axis_index("s")
    wid = sid * NC + cid

    def zrow_u(j, _):
        for cc in range(C // LANES):
            ub[j, pl.ds(cc * LANES, LANES)] = jnp.zeros((LANES,), jnp.float32)
        return 0

    lax.fori_loop(0, EK, zrow_u, 0)

    def zrow_c(j, _):
        cb[j, :] = jnp.zeros((LANES,), jnp.float32)
        return 0

    lax.fori_loop(0, EK, zrow_c, 0)

    nbase = sid * NR_PER_T
    for joff in range(0, NR_PER_T, EK):
        size = min(EK, NR_PER_T - joff)
        pltpu.sync_copy(ub.at[pl.ds(0, size)],
                        sshared.at[pl.ds(nbase + joff, size)])
        pltpu.sync_copy(cb.at[pl.ds(0, size)],
                        fshared.at[pl.ds(nbase + joff, size)])
    plsc.subcore_barrier()

    base = wid * E_PER_W
    iot = lax.broadcasted_iota(jnp.int32, (LANES,), 0)

    def chunk(i, _):
        off = base + i * EK
        pltpu.sync_copy(src_hbm.at[pl.ds(off, EK)], sidx)
        pltpu.sync_copy(dst_hbm.at[pl.ds(off, EK)], didx)
        pltpu.async_copy(x134_hbm.at[sidx], xsb, sem).wait()
        pltpu.sync_copy(u_hbm.at[pl.ds(off, EK)], ub)

        def edge_dot(k, _):
            z = jnp.zeros((LANES,), jnp.float32)
            a0, a1, a2 = z, z, z
            for j in range(C // LANES):
                sl = pl.ds(j * LANES, LANES)
                uv = ub[k, sl]
                a0 = a0 + xsb[k, pl.ds(j * LANES, LANES)] * uv
                a1 = a1 + xsb[k, pl.ds(C + j * LANES, LANES)] * uv
                a2 = a2 + xsb[k, pl.ds(2 * C + j * LANES, LANES)] * uv
            s0 = jnp.sum(a0)
            s1 = jnp.sum(a1)
            s2 = jnp.sum(a2)
            cvec = (jnp.where(iot == 0, s0, 0.0)
                    + jnp.where(iot == 1, s1, 0.0)
                    + jnp.where(iot == 2, s2, 0.0))
            cb[k, :] = cvec
            return 0

        lax.fori_loop(0, EK, edge_dot, 0)
        pltpu.sync_copy(cb, fshared.at[didx], add=True)
        pltpu.sync_copy(ub, sshared.at[didx], add=True)
        return 0

    lax.fori_loop(0, N_CHUNKS, chunk, 0)
    plsc.subcore_barrier()

    pltpu.sync_copy(fshared.at[pl.ds(nbase, NR_PER_T)],
                    fsrc_hbm.at[cid, pl.ds(nbase, NR_PER_T)])
    pltpu.sync_copy(sshared.at[pl.ds(nbase, NR_PER_T)],
                    ssum_hbm.at[cid, pl.ds(nbase, NR_PER_T)])


def _src_dots(x134, u, src, dst):
    f = pl.kernel(
        _src_body,
        out_type=[
            jax.ShapeDtypeStruct((NC, N_N, 16), jnp.float32),
            jax.ShapeDtypeStruct((NC, N_N, C), jnp.float32),
        ],
        mesh=plsc.VectorSubcoreMesh(**_SC_MESH),
        scratch_types=[
            pltpu.VMEM((EK,), jnp.int32),
            pltpu.VMEM((EK,), jnp.int32),
            pltpu.VMEM((EK, 3 * C), jnp.float32),
            pltpu.VMEM((EK, C), jnp.float32),
            pltpu.VMEM((EK, 16), jnp.float32),
            pltpu.VMEM_SHARED((N_N, 16), jnp.float32),
            pltpu.VMEM_SHARED((N_N, C), jnp.float32),
            pltpu.SemaphoreType.DMA,
        ],
    )
    return f(x134, u, src, dst)


def _final_body(fsrc_ref, s_ref, x134_ref, out_ref):
    s = s_ref[0] + s_ref[1]
    f = fsrc_ref[0] + fsrc_ref[1]
    bn = f.shape[0]
    iot = lax.broadcasted_iota(jnp.int32, (bn, 16), 1)
    acc = f
    for l in range(3):
        d = jnp.sum(x134_ref[...][:, l * C:(l + 1) * C] * s, axis=1,
                    keepdims=True)
        acc = acc + jnp.where(iot == l, d, 0.0)
    out_ref[...] = acc


def _finalize(fsrc, ssum, x134):
    bn = 1000
    return pl.pallas_call(
        _final_body,
        grid=(N_N // bn,),
        in_specs=[
            pl.BlockSpec((NC, bn, 16), lambda i: (0, i, 0)),
            pl.BlockSpec((NC, bn, C), lambda i: (0, i, 0)),
            pl.BlockSpec((bn, 3 * C), lambda i: (i, 0)),
        ],
        out_specs=pl.BlockSpec((bn, 16), lambda i: (i, 0)),
        out_shape=jax.ShapeDtypeStruct((N_N, 16), jnp.float32),
    )(fsrc, ssum, x134)


@jax.jit
def _run(x, atomic_numbers, edge_distance, edge_index, node_offset, p):
    src = edge_index[0].astype(jnp.int32)
    dst = (edge_index[1] - node_offset).astype(jnp.int32)

    an2 = atomic_numbers.astype(jnp.int32).reshape(N_N, 1)
    x0 = x[:, 0, :]
    x134 = x[:, 1:4, :].reshape(N_N, 3 * C)
    asrc_pad = jnp.zeros((128, C), jnp.float32).at[:90].set(p["atom_src"])
    adst_pad = jnp.zeros((128, C), jnp.float32).at[:90].set(p["atom_dst"])

    avflat = p["alpha_vec"].reshape(1, H * A)
    k_i = jnp.arange(H * A)[:, None] // A
    sel = (k_i == jnp.arange(16)[None, :]).astype(jnp.float32)
    rep = (jnp.arange(H)[:, None] == (jnp.arange(H * V)[None, :] // V)
           ).astype(jnp.float32)
    wvt = p["Wv"].T
    wo_row = p["Wo"][:, 0].reshape(1, H * V)
    wa1s = p["Wa1"][:C]
    wa1r = p["Wa1"][C:]

    u_tab, w_tab = _build_tables(an2, x0, asrc_pad, adst_pad)
    embcat = _edge_gather(u_tab, w_tab, src, dst)
    exw, gw = _edge_math(embcat, edge_distance, p["W_rad1"], p["W_rad2"],
                         wa1s, wa1r, avflat, sel, p["Wg"], wo_row)
    deng = _den_scatter_gather(exw, dst)
    u = _u_project(exw, deng, gw, rep, wvt)
    fsrc, ssum = _src_dots(x134, u, src, dst)
    out16 = _finalize(fsrc, ssum, x134)
    return out16[:, :3]


def kernel(x, atomic_numbers, edge_distance, edge_index, node_offset,
           params_force, params_denoise):
    del params_denoise
    return _run(x, atomic_numbers, edge_distance, edge_index,
                jnp.asarray(node_offset, jnp.int32), params_force)


# speedup vs baseline: 31.3688x; 31.3688x over previous
import jax
import jax.numpy as jnp
from jax import lax
from jax.experimental import pallas as pl
from jax.experimental.pallas import tpu as pltpu
from jax.experimental.pallas import tpu_sc as plsc

N_N = 10000
N_P = 10240
E_E = 160000
C = 128
H = 8
V = 16
A = 64
R = 64

NC = 2
NS = 16
NW = NC * NS
LANES = 16

CK = 128
TOT_CHUNKS = E_E // CK
RR_ITERS = (TOT_CHUNKS + NW - 1) // NW
CK5 = 40
TOT5 = E_E // CK5
RR5 = (TOT5 + NW - 1) // NW

_SC_MESH = dict(core_axis_name="c", subcore_axis_name="s")


def _tables_body(an_ref, x0_ref, asrc_ref, adst_ref, u_ref, w_ref):
    an = an_ref[...]
    iot = lax.broadcasted_iota(jnp.int32, (1, 128), 1)
    onehot = (an == iot).astype(jnp.float32)
    tsrc = jnp.dot(onehot, asrc_ref[...], preferred_element_type=jnp.float32)
    tdst = jnp.dot(onehot, adst_ref[...], preferred_element_type=jnp.float32)
    x0 = x0_ref[...]
    u_ref[...] = jnp.concatenate([tsrc, x0], axis=1)
    w_ref[...] = jnp.concatenate([tdst, x0], axis=1)


def _build_tables(an2, x0, asrc_pad, adst_pad):
    bn = 1024
    return pl.pallas_call(
        _tables_body,
        grid=(N_P // bn,),
        in_specs=[
            pl.BlockSpec((bn, 1), lambda i: (i, 0)),
            pl.BlockSpec((bn, C), lambda i: (i, 0)),
            pl.BlockSpec((128, C), lambda i: (0, 0)),
            pl.BlockSpec((128, C), lambda i: (0, 0)),
        ],
        out_specs=[
            pl.BlockSpec((bn, 2 * C), lambda i: (i, 0)),
            pl.BlockSpec((bn, 2 * C), lambda i: (i, 0)),
        ],
        out_shape=[
            jax.ShapeDtypeStruct((N_P, 2 * C), jnp.float32),
            jax.ShapeDtypeStruct((N_P, 2 * C), jnp.float32),
        ],
    )(an2, x0, asrc_pad, adst_pad)


def _edge_gather_body(u_hbm, w_hbm, src_hbm, dst_hbm, out_hbm,
                      sidx, didx, rows_u, rows_w, sem1, sem2):
    wid = lax.axis_index("s") * NC + lax.axis_index("c")

    def chunk(t, _):
        cidx = wid + NW * t

        @pl.when(cidx < TOT_CHUNKS)
        def _():
            off = cidx * CK
            pltpu.sync_copy(src_hbm.at[pl.ds(off, CK)], sidx)
            pltpu.sync_copy(dst_hbm.at[pl.ds(off, CK)], didx)
            cu = pltpu.async_copy(u_hbm.at[sidx], rows_u, sem1)
            cw = pltpu.async_copy(w_hbm.at[didx], rows_w, sem2)
            cu.wait()
            cw.wait()

            def addrow(k, _):
                for cc in range(2 * C // LANES):
                    sl = pl.ds(cc * LANES, LANES)
                    rows_u[k, sl] = rows_u[k, sl] + rows_w[k, sl]
                return 0

            lax.fori_loop(0, CK, addrow, 0)
            pltpu.sync_copy(rows_u, out_hbm.at[pl.ds(off, CK)])

        return 0

    lax.fori_loop(0, RR_ITERS, chunk, 0)


def _edge_gather(u_tab, w_tab, src, dst):
    f = pl.kernel(
        _edge_gather_body,
        out_type=jax.ShapeDtypeStruct((E_E, 2 * C), jnp.float32),
        mesh=plsc.VectorSubcoreMesh(**_SC_MESH),
        scratch_types=[
            pltpu.VMEM((CK,), jnp.int32),
            pltpu.VMEM((CK,), jnp.int32),
            pltpu.VMEM((CK, 2 * C), jnp.float32),
            pltpu.VMEM((CK, 2 * C), jnp.float32),
            pltpu.SemaphoreType.DMA,
            pltpu.SemaphoreType.DMA,
        ],
    )
    return f(u_tab, w_tab, src, dst)


def _edge_math_body(embcat_ref, ed_ref, wr1_ref, wr2_ref, wa1s_ref, wa1r_ref,
                    avflat_ref, sel_ref, wg_ref, wo_ref, exw_ref, gw_ref):
    embcat = embcat_ref[...]
    emb = embcat[:, :C] + ed_ref[...]
    scalar = embcat[:, C:]
    rad = jax.nn.silu(jnp.dot(emb, wr1_ref[...],
                              preferred_element_type=jnp.float32))
    rad = jax.nn.silu(jnp.dot(rad, wr2_ref[...],
                              preferred_element_type=jnp.float32))
    a = jax.nn.silu(
        jnp.dot(scalar, wa1s_ref[...], preferred_element_type=jnp.float32)
        + jnp.dot(rad, wa1r_ref[...], preferred_element_type=jnp.float32))
    aw = a * avflat_ref[...]
    logits128 = jnp.dot(aw, sel_ref[...], preferred_element_type=jnp.float32)
    exw_ref[...] = jnp.exp(logits128)
    gate = jax.nn.sigmoid(jnp.dot(rad, wg_ref[...],
                                  preferred_element_type=jnp.float32))
    gw_ref[...] = gate * wo_ref[...]


def _edge_math(embcat, edge_distance, wr1, wr2, wa1s, wa1r, avflat, sel, wg,
               wo_row):
    be = 2000
    return pl.pallas_call(
        _edge_math_body,
        grid=(E_E // be,),
        in_specs=[
            pl.BlockSpec((be, 2 * C), lambda i: (i, 0)),
            pl.BlockSpec((be, C), lambda i: (i, 0)),
            pl.BlockSpec((C, R), lambda i: (0, 0)),
            pl.BlockSpec((R, R), lambda i: (0, 0)),
            pl.BlockSpec((C, H * A), lambda i: (0, 0)),
            pl.BlockSpec((R, H * A), lambda i: (0, 0)),
            pl.BlockSpec((1, H * A), lambda i: (0, 0)),
            pl.BlockSpec((H * A, C), lambda i: (0, 0)),
            pl.BlockSpec((R, H * V), lambda i: (0, 0)),
            pl.BlockSpec((1, H * V), lambda i: (0, 0)),
        ],
        out_specs=[
            pl.BlockSpec((be, C), lambda i: (i, 0)),
            pl.BlockSpec((be, H * V), lambda i: (i, 0)),
        ],
        out_shape=[
            jax.ShapeDtypeStruct((E_E, C), jnp.float32),
            jax.ShapeDtypeStruct((E_E, H * V), jnp.float32),
        ],
    )(embcat, edge_distance, wr1, wr2, wa1s, wa1r, avflat, sel, wg, wo_row)


def _den_body(zeros_hbm, exw_hbm, dst_hbm, denp_hbm, idxb, rowsb, dshared):
    cid = lax.axis_index("c")
    sid = lax.axis_index("s")
    wid = sid * NC + cid

    @pl.when(sid == 0)
    def _():
        pltpu.sync_copy(zeros_hbm, dshared)

    plsc.subcore_barrier()

    def chunk(t, _):
        cidx = wid + NW * t

        @pl.when(cidx < TOT_CHUNKS)
        def _():
            off = cidx * CK
            pltpu.sync_copy(dst_hbm.at[pl.ds(off, CK)], idxb.at[0])
            pltpu.sync_copy(exw_hbm.at[pl.ds(off, CK)], rowsb)
            pltpu.sync_copy(rowsb, dshared.at[idxb.at[0]], add=True)

        return 0

    lax.fori_loop(0, RR_ITERS, chunk, 0)
    plsc.subcore_barrier()

    @pl.when(sid == 0)
    def _():
        pltpu.sync_copy(dshared, denp_hbm.at[cid])


def _den_scatter(zerosc, exw, dst):
    f = pl.kernel(
        _den_body,
        out_type=jax.ShapeDtypeStruct((NC, N_P, C), jnp.float32),
        mesh=plsc.VectorSubcoreMesh(**_SC_MESH),
        scratch_types=[
            pltpu.VMEM((1, CK), jnp.int32),
            pltpu.VMEM((CK, C), jnp.float32),
            pltpu.VMEM_SHARED((N_P, C), jnp.float32),
        ],
    )
    return f(zerosc, exw, dst)


def _denc_body(denp_ref, den_ref):
    den_ref[...] = denp_ref[0] + denp_ref[1]


def _den_combine(denp):
    bn = 1024
    return pl.pallas_call(
        _denc_body,
        grid=(N_P // bn,),
        in_specs=[pl.BlockSpec((NC, bn, C), lambda i: (0, i, 0))],
        out_specs=pl.BlockSpec((bn, C), lambda i: (i, 0)),
        out_shape=jax.ShapeDtypeStruct((N_P, C), jnp.float32),
    )(denp)


def _deng_body(den_hbm, dst_hbm, deng_hbm, idxb, rows128, rowsb, sem):
    wid = lax.axis_index("s") * NC + lax.axis_index("c")

    def gchunk(t, _):
        cidx = wid + NW * t

        @pl.when(cidx < TOT_CHUNKS)
        def _():
            off = cidx * CK
            pltpu.sync_copy(dst_hbm.at[pl.ds(off, CK)], idxb)
            pltpu.async_copy(den_hbm.at[idxb], rows128, sem).wait()

            def narrow(k, _):
                rowsb[k, :] = rows128[k, pl.ds(0, LANES)]
                return 0

            lax.fori_loop(0, CK, narrow, 0)
            pltpu.sync_copy(rowsb, deng_hbm.at[pl.ds(off, CK)])

        return 0

    lax.fori_loop(0, RR_ITERS, gchunk, 0)


def _den_gather(den, dst):
    f = pl.kernel(
        _deng_body,
        out_type=jax.ShapeDtypeStruct((E_E, 16), jnp.float32),
        mesh=plsc.VectorSubcoreMesh(**_SC_MESH),
        scratch_types=[
            pltpu.VMEM((CK,), jnp.int32),
            pltpu.VMEM((CK, C), jnp.float32),
            pltpu.VMEM((CK, 16), jnp.float32),
            pltpu.SemaphoreType.DMA,
        ],
    )
    return f(den, dst)


def _u_body(exw_ref, deng_ref, gw_ref, rep_ref, wvt_ref, u_ref):
    ex = exw_ref[...][:, :H]
    den = deng_ref[...][:, :H]
    alpha = ex / (den + 1e-9)
    alpha_rep = jnp.dot(alpha, rep_ref[...], preferred_element_type=jnp.float32)
    u_ref[...] = jnp.dot(alpha_rep * gw_ref[...], wvt_ref[...],
                         preferred_element_type=jnp.float32)


def _u_project(exw, deng, gw, rep, wvt):
    be = 2000
    return pl.pallas_call(
        _u_body,
        grid=(E_E // be,),
        in_specs=[
            pl.BlockSpec((be, C), lambda i: (i, 0)),
            pl.BlockSpec((be, 16), lambda i: (i, 0)),
            pl.BlockSpec((be, H * V), lambda i: (i, 0)),
            pl.BlockSpec((H, H * V), lambda i: (0, 0)),
            pl.BlockSpec((H * V, C), lambda i: (0, 0)),
        ],
        out_specs=pl.BlockSpec((be, C), lambda i: (i, 0)),
        out_shape=jax.ShapeDtypeStruct((E_E, C), jnp.float32),
    )(exw, deng, gw, rep, wvt)


def _src_body(zeros_hbm, x134_hbm, u_hbm, src_hbm, dst_hbm, fsrc_hbm,
              sidx, didx, xsb, ub, cb, fshared, sem):
    cid = lax.axis_index("c")
    sid = lax.axis_index("s")
    wid = sid * NC + cid

    @pl.when(sid == 0)
    def _():
        pltpu.sync_copy(zeros_hbm, fshared)

    def zrow_c(j, _):
        for cc in range(C // LANES):
            cb[j, pl.ds(cc * LANES, LANES)] = jnp.zeros((LANES,), jnp.float32)
        return 0

    lax.fori_loop(0, CK5, zrow_c, 0)
    plsc.subcore_barrier()

    def chunk(t, _):
        cidx = wid + NW * t

        @pl.when(cidx < TOT5)
        def _():
            off = cidx * CK5
            pltpu.sync_copy(src_hbm.at[pl.ds(off, CK5)], sidx)
            pltpu.sync_copy(dst_hbm.at[pl.ds(off, CK5)], didx.at[0])
            pltpu.async_copy(x134_hbm.at[sidx], xsb, sem).wait()
            pltpu.sync_copy(u_hbm.at[pl.ds(off, CK5)], ub)

            def edge_dot(k, _):
                z = jnp.zeros((LANES,), jnp.float32)
                a0, a1, a2 = z, z, z
                for j in range(C // LANES):
                    sl = pl.ds(j * LANES, LANES)
                    uv = ub[k, sl]
                    a0 = a0 + xsb[k, pl.ds(j * LANES, LANES)] * uv
                    a1 = a1 + xsb[k, pl.ds(C + j * LANES, LANES)] * uv
                    a2 = a2 + xsb[k, pl.ds(2 * C + j * LANES, LANES)] * uv
                cb[k, pl.ds(0, LANES)] = a0
                cb[k, pl.ds(LANES, LANES)] = a1
                cb[k, pl.ds(2 * LANES, LANES)] = a2
                return 0

            lax.fori_loop(0, CK5, edge_dot, 0)
            pltpu.sync_copy(cb, fshared.at[didx.at[0]], add=True)

        return 0

    lax.fori_loop(0, RR5, chunk, 0)
    plsc.subcore_barrier()

    @pl.when(sid == 0)
    def _():
        pltpu.sync_copy(fshared, fsrc_hbm.at[cid])


def _src_dots(zerosc, x134, u, src, dst):
    f = pl.kernel(
        _src_body,
        out_type=jax.ShapeDtypeStruct((NC, N_P, C), jnp.float32),
        mesh=plsc.VectorSubcoreMesh(**_SC_MESH),
        scratch_types=[
            pltpu.VMEM((CK5,), jnp.int32),
            pltpu.VMEM((1, CK5), jnp.int32),
            pltpu.VMEM((CK5, 3 * C), jnp.float32),
            pltpu.VMEM((CK5, C), jnp.float32),
            pltpu.VMEM((CK5, C), jnp.float32),
            pltpu.VMEM_SHARED((N_P, C), jnp.float32),
            pltpu.SemaphoreType.DMA,
        ],
    )
    return f(zerosc, x134, u, src, dst)


def _usum_body(zeros_hbm, u_hbm, dst_hbm, ssum_hbm, didx, ub, sshared):
    cid = lax.axis_index("c")
    sid = lax.axis_index("s")
    wid = sid * NC + cid

    @pl.when(sid == 0)
    def _():
        pltpu.sync_copy(zeros_hbm, sshared)

    plsc.subcore_barrier()

    def chunk(t, _):
        cidx = wid + NW * t

        @pl.when(cidx < TOT_CHUNKS)
        def _():
            off = cidx * CK
            pltpu.sync_copy(dst_hbm.at[pl.ds(off, CK)], didx.at[0])
            pltpu.sync_copy(u_hbm.at[pl.ds(off, CK)], ub)
            pltpu.sync_copy(ub, sshared.at[didx.at[0]], add=True)

        return 0

    lax.fori_loop(0, RR_ITERS, chunk, 0)
    plsc.subcore_barrier()

    @pl.when(sid == 0)
    def _():
        pltpu.sync_copy(sshared, ssum_hbm.at[cid])


def _u_scatter(zerosc, u, dst):
    f = pl.kernel(
        _usum_body,
        out_type=jax.ShapeDtypeStruct((NC, N_P, C), jnp.float32),
        mesh=plsc.VectorSubcoreMesh(**_SC_MESH),
        scratch_types=[
            pltpu.VMEM((1, CK), jnp.int32),
            pltpu.VMEM((CK, C), jnp.float32),
            pltpu.VMEM_SHARED((N_P, C), jnp.float32),
        ],
    )
    return f(zerosc, u, dst)


def _final_body(fsrc_ref, s_ref, x134_ref, out_ref):
    s = s_ref[0] + s_ref[1]
    f128 = fsrc_ref[0] + fsrc_ref[1]
    bn = f128.shape[0]
    iot = lax.broadcasted_iota(jnp.int32, (bn, 16), 1)
    acc = jnp.zeros((bn, 16), jnp.float32)
    for l in range(3):
        d = jnp.sum(x134_ref[...][:, l * C:(l + 1) * C] * s, axis=1,
                    keepdims=True)
        d = d + jnp.sum(f128[:, l * 16:(l + 1) * 16], axis=1, keepdims=True)
        acc = acc + jnp.where(iot == l, d, 0.0)
    out_ref[...] = acc


def _finalize(fsrc, ssum, x134):
    bn = 1024
    return pl.pallas_call(
        _final_body,
        grid=(N_P // bn,),
        in_specs=[
            pl.BlockSpec((NC, bn, C), lambda i: (0, i, 0)),
            pl.BlockSpec((NC, bn, C), lambda i: (0, i, 0)),
            pl.BlockSpec((bn, 3 * C), lambda i: (i, 0)),
        ],
        out_specs=pl.BlockSpec((bn, 16), lambda i: (i, 0)),
        out_shape=jax.ShapeDtypeStruct((N_P, 16), jnp.float32),
    )(fsrc, ssum, x134)


@jax.jit
def _run(x, atomic_numbers, edge_distance, edge_index, node_offset, p):
    src = edge_index[0].astype(jnp.int32)
    dst = (edge_index[1] - node_offset).astype(jnp.int32)

    an2 = jnp.full((N_P, 1), 127, jnp.int32).at[:N_N, 0].set(
        atomic_numbers.astype(jnp.int32))
    x0 = jnp.zeros((N_P, C), jnp.float32).at[:N_N].set(x[:, 0, :])
    x134 = jnp.zeros((N_P, 3 * C), jnp.float32).at[:N_N].set(
        x[:, 1:4, :].reshape(N_N, 3 * C))
    asrc_pad = jnp.zeros((128, C), jnp.float32).at[:90].set(p["atom_src"])
    adst_pad = jnp.zeros((128, C), jnp.float32).at[:90].set(p["atom_dst"])
    zerosc = jnp.zeros((N_P, C), jnp.float32)

    avflat = p["alpha_vec"].reshape(1, H * A)
    k_i = jnp.arange(H * A)[:, None] // A
    sel = (k_i == jnp.arange(C)[None, :]).astype(jnp.float32)
    rep = (jnp.arange(H)[:, None] == (jnp.arange(H * V)[None, :] // V)
           ).astype(jnp.float32)
    wvt = p["Wv"].T
    wo_row = p["Wo"][:, 0].reshape(1, H * V)
    wa1s = p["Wa1"][:C]
    wa1r = p["Wa1"][C:]

    u_tab, w_tab = _build_tables(an2, x0, asrc_pad, adst_pad)
    embcat = _edge_gather(u_tab, w_tab, src, dst)
    exw, gw = _edge_math(embcat, edge_distance, p["W_rad1"], p["W_rad2"],
                         wa1s, wa1r, avflat, sel, p["Wg"], wo_row)
    denp = _den_scatter(zerosc, exw, dst)
    den = _den_combine(denp)
    deng = _den_gather(den, dst)
    u = _u_project(exw, deng, gw, rep, wvt)
    fsrc = _src_dots(zerosc, x134, u, src, dst)
    ssum = _u_scatter(zerosc, u, dst)
    out16 = _finalize(fsrc, ssum, x134)
    return out16[:N_N, :3]


def kernel(x, atomic_numbers, edge_distance, edge_index, node_offset,
           params_force, params_denoise):
    del params_denoise
    return _run(x, atomic_numbers, edge_distance, edge_index,
                jnp.asarray(node_offset, jnp.int32), params_force)


# speedup vs baseline: 34.2886x; 1.0931x over previous
import jax
import jax.numpy as jnp
from jax import lax
from jax.experimental import pallas as pl
from jax.experimental.pallas import tpu as pltpu
from jax.experimental.pallas import tpu_sc as plsc

N_N = 10000
N_P = 10240
E_E = 160000
C = 128
H = 8
V = 16
A = 64
R = 64

NC = 2
NS = 16
NW = NC * NS
LANES = 16

CK = 128
TOT_CHUNKS = E_E // CK
RR_ITERS = (TOT_CHUNKS + NW - 1) // NW
CK5 = 64
TOT5 = E_E // CK5
RR5 = (TOT5 + NW - 1) // NW

_SC_MESH = dict(core_axis_name="c", subcore_axis_name="s")


def _tables_body(an_ref, x0_ref, asrc_ref, adst_ref, u_ref, w_ref):
    an = an_ref[...]
    iot = lax.broadcasted_iota(jnp.int32, (1, 128), 1)
    onehot = (an == iot).astype(jnp.float32)
    tsrc = jnp.dot(onehot, asrc_ref[...], preferred_element_type=jnp.float32)
    tdst = jnp.dot(onehot, adst_ref[...], preferred_element_type=jnp.float32)
    x0 = x0_ref[...]
    u_ref[...] = jnp.concatenate([tsrc, x0], axis=1)
    w_ref[...] = jnp.concatenate([tdst, x0], axis=1)


def _build_tables(an2, x0, asrc_pad, adst_pad):
    bn = 1024
    return pl.pallas_call(
        _tables_body,
        grid=(N_P // bn,),
        in_specs=[
            pl.BlockSpec((bn, 1), lambda i: (i, 0)),
            pl.BlockSpec((bn, C), lambda i: (i, 0)),
            pl.BlockSpec((128, C), lambda i: (0, 0)),
            pl.BlockSpec((128, C), lambda i: (0, 0)),
        ],
        out_specs=[
            pl.BlockSpec((bn, 2 * C), lambda i: (i, 0)),
            pl.BlockSpec((bn, 2 * C), lambda i: (i, 0)),
        ],
        out_shape=[
            jax.ShapeDtypeStruct((N_P, 2 * C), jnp.float32),
            jax.ShapeDtypeStruct((N_P, 2 * C), jnp.float32),
        ],
    )(an2, x0, asrc_pad, adst_pad)


def _edge_gather_body(u_hbm, w_hbm, src_hbm, dst_hbm, out_hbm,
                      sidx, didx, rows_u, rows_w, sem1, sem2):
    wid = lax.axis_index("s") * NC + lax.axis_index("c")

    def chunk(t, _):
        cidx = wid + NW * t

        @pl.when(cidx < TOT_CHUNKS)
        def _():
            off = cidx * CK
            pltpu.sync_copy(src_hbm.at[pl.ds(off, CK)], sidx)
            pltpu.sync_copy(dst_hbm.at[pl.ds(off, CK)], didx)
            cu = pltpu.async_copy(u_hbm.at[sidx], rows_u, sem1)
            cw = pltpu.async_copy(w_hbm.at[didx], rows_w, sem2)
            cu.wait()
            cw.wait()

            def addrow(k, _):
                for cc in range(2 * C // LANES):
                    sl = pl.ds(cc * LANES, LANES)
                    rows_u[k, sl] = rows_u[k, sl] + rows_w[k, sl]
                return 0

            lax.fori_loop(0, CK, addrow, 0)
            pltpu.sync_copy(rows_u, out_hbm.at[pl.ds(off, CK)])

        return 0

    lax.fori_loop(0, RR_ITERS, chunk, 0)


def _edge_gather(u_tab, w_tab, src, dst):
    f = pl.kernel(
        _edge_gather_body,
        out_type=jax.ShapeDtypeStruct((E_E, 2 * C), jnp.float32),
        mesh=plsc.VectorSubcoreMesh(**_SC_MESH),
        scratch_types=[
            pltpu.VMEM((CK,), jnp.int32),
            pltpu.VMEM((CK,), jnp.int32),
            pltpu.VMEM((CK, 2 * C), jnp.float32),
            pltpu.VMEM((CK, 2 * C), jnp.float32),
            pltpu.SemaphoreType.DMA,
            pltpu.SemaphoreType.DMA,
        ],
    )
    return f(u_tab, w_tab, src, dst)


def _edge_math_body(embcat_ref, ed_ref, wr1_ref, wr2_ref, wa1s_ref, wa1r_ref,
                    avflat_ref, sel_ref, wg_ref, wo_ref, exw_ref, gw_ref):
    embcat = embcat_ref[...]
    emb = embcat[:, :C] + ed_ref[...]
    scalar = embcat[:, C:]
    rad = jax.nn.silu(jnp.dot(emb, wr1_ref[...],
                              preferred_element_type=jnp.float32))
    rad = jax.nn.silu(jnp.dot(rad, wr2_ref[...],
                              preferred_element_type=jnp.float32))
    a = jax.nn.silu(
        jnp.dot(scalar, wa1s_ref[...], preferred_element_type=jnp.float32)
        + jnp.dot(rad, wa1r_ref[...], preferred_element_type=jnp.float32))
    aw = a * avflat_ref[...]
    logits128 = jnp.dot(aw, sel_ref[...], preferred_element_type=jnp.float32)
    exw_ref[...] = jnp.exp(logits128)
    gate = jax.nn.sigmoid(jnp.dot(rad, wg_ref[...],
                                  preferred_element_type=jnp.float32))
    gw_ref[...] = gate * wo_ref[...]


def _edge_math(embcat, edge_distance, wr1, wr2, wa1s, wa1r, avflat, sel, wg,
               wo_row):
    be = 2000
    return pl.pallas_call(
        _edge_math_body,
        grid=(E_E // be,),
        in_specs=[
            pl.BlockSpec((be, 2 * C), lambda i: (i, 0)),
            pl.BlockSpec((be, C), lambda i: (i, 0)),
            pl.BlockSpec((C, R), lambda i: (0, 0)),
            pl.BlockSpec((R, R), lambda i: (0, 0)),
            pl.BlockSpec((C, H * A), lambda i: (0, 0)),
            pl.BlockSpec((R, H * A), lambda i: (0, 0)),
            pl.BlockSpec((1, H * A), lambda i: (0, 0)),
            pl.BlockSpec((H * A, C), lambda i: (0, 0)),
            pl.BlockSpec((R, H * V), lambda i: (0, 0)),
            pl.BlockSpec((1, H * V), lambda i: (0, 0)),
        ],
        out_specs=[
            pl.BlockSpec((be, C), lambda i: (i, 0)),
            pl.BlockSpec((be, H * V), lambda i: (i, 0)),
        ],
        out_shape=[
            jax.ShapeDtypeStruct((E_E, C), jnp.float32),
            jax.ShapeDtypeStruct((E_E, H * V), jnp.float32),
        ],
    )(embcat, edge_distance, wr1, wr2, wa1s, wa1r, avflat, sel, wg, wo_row)


def _den_body(zeros_hbm, exw_hbm, dst_hbm, denp_hbm, idxb, rowsb, dshared):
    cid = lax.axis_index("c")
    sid = lax.axis_index("s")
    wid = sid * NC + cid

    @pl.when(sid == 0)
    def _():
        pltpu.sync_copy(zeros_hbm, dshared)

    plsc.subcore_barrier()

    def chunk(t, _):
        cidx = wid + NW * t

        @pl.when(cidx < TOT_CHUNKS)
        def _():
            off = cidx * CK
            pltpu.sync_copy(dst_hbm.at[pl.ds(off, CK)], idxb.at[0])
            pltpu.sync_copy(exw_hbm.at[pl.ds(off, CK)], rowsb)
            pltpu.sync_copy(rowsb, dshared.at[idxb.at[0]], add=True)

        return 0

    lax.fori_loop(0, RR_ITERS, chunk, 0)
    plsc.subcore_barrier()

    @pl.when(sid == 0)
    def _():
        pltpu.sync_copy(dshared, denp_hbm.at[cid])


def _den_scatter(zerosc, exw, dst):
    f = pl.kernel(
        _den_body,
        out_type=jax.ShapeDtypeStruct((NC, N_P, C), jnp.float32),
        mesh=plsc.VectorSubcoreMesh(**_SC_MESH),
        scratch_types=[
            pltpu.VMEM((1, CK), jnp.int32),
            pltpu.VMEM((CK, C), jnp.float32),
            pltpu.VMEM_SHARED((N_P, C), jnp.float32),
        ],
    )
    return f(zerosc, exw, dst)


def _denc_body(denp_ref, den_ref):
    den_ref[...] = denp_ref[0] + denp_ref[1]


def _den_combine(denp):
    bn = 1024
    return pl.pallas_call(
        _denc_body,
        grid=(N_P // bn,),
        in_specs=[pl.BlockSpec((NC, bn, C), lambda i: (0, i, 0))],
        out_specs=pl.BlockSpec((bn, C), lambda i: (i, 0)),
        out_shape=jax.ShapeDtypeStruct((N_P, C), jnp.float32),
    )(denp)


def _deng_body(den_hbm, dst_hbm, deng_hbm, idxb, rows128, rowsb, sem):
    wid = lax.axis_index("s") * NC + lax.axis_index("c")

    def gchunk(t, _):
        cidx = wid + NW * t

        @pl.when(cidx < TOT_CHUNKS)
        def _():
            off = cidx * CK
            pltpu.sync_copy(dst_hbm.at[pl.ds(off, CK)], idxb)
            pltpu.async_copy(den_hbm.at[idxb], rows128, sem).wait()

            def narrow(k, _):
                rowsb[k, :] = rows128[k, pl.ds(0, LANES)]
                return 0

            lax.fori_loop(0, CK, narrow, 0)
            pltpu.sync_copy(rowsb, deng_hbm.at[pl.ds(off, CK)])

        return 0

    lax.fori_loop(0, RR_ITERS, gchunk, 0)


def _den_gather(den, dst):
    f = pl.kernel(
        _deng_body,
        out_type=jax.ShapeDtypeStruct((E_E, 16), jnp.float32),
        mesh=plsc.VectorSubcoreMesh(**_SC_MESH),
        scratch_types=[
            pltpu.VMEM((CK,), jnp.int32),
            pltpu.VMEM((CK, C), jnp.float32),
            pltpu.VMEM((CK, 16), jnp.float32),
            pltpu.SemaphoreType.DMA,
        ],
    )
    return f(den, dst)


def _u_body(exw_ref, deng_ref, gw_ref, rep_ref, wvt_ref, u_ref):
    ex = exw_ref[...][:, :H]
    den = deng_ref[...][:, :H]
    alpha = ex / (den + 1e-9)
    alpha_rep = jnp.dot(alpha, rep_ref[...], preferred_element_type=jnp.float32)
    u_ref[...] = jnp.dot(alpha_rep * gw_ref[...], wvt_ref[...],
                         preferred_element_type=jnp.float32)


def _u_project(exw, deng, gw, rep, wvt):
    be = 2000
    return pl.pallas_call(
        _u_body,
        grid=(E_E // be,),
        in_specs=[
            pl.BlockSpec((be, C), lambda i: (i, 0)),
            pl.BlockSpec((be, 16), lambda i: (i, 0)),
            pl.BlockSpec((be, H * V), lambda i: (i, 0)),
            pl.BlockSpec((H, H * V), lambda i: (0, 0)),
            pl.BlockSpec((H * V, C), lambda i: (0, 0)),
        ],
        out_specs=pl.BlockSpec((be, C), lambda i: (i, 0)),
        out_shape=jax.ShapeDtypeStruct((E_E, C), jnp.float32),
    )(exw, deng, gw, rep, wvt)


def _src_body(zeros_hbm, x134_hbm, u_hbm, src_hbm, dst_hbm, fsrc_hbm,
              sidx, didx, xsb, ub, cb, fshared, sem):
    cid = lax.axis_index("c")
    sid = lax.axis_index("s")
    wid = sid * NC + cid

    @pl.when(sid == 0)
    def _():
        pltpu.sync_copy(zeros_hbm, fshared)

    def zrow_c(j, _):
        for cc in range(C // LANES):
            cb[j, pl.ds(cc * LANES, LANES)] = jnp.zeros((LANES,), jnp.float32)
        return 0

    lax.fori_loop(0, CK5, zrow_c, 0)
    plsc.subcore_barrier()

    def chunk(t, _):
        cidx = wid + NW * t

        @pl.when(cidx < TOT5)
        def _():
            off = cidx * CK5
            pltpu.sync_copy(src_hbm.at[pl.ds(off, CK5)], sidx)
            pltpu.sync_copy(dst_hbm.at[pl.ds(off, CK5)], didx.at[0])
            cx = pltpu.async_copy(x134_hbm.at[sidx], xsb, sem)
            pltpu.sync_copy(u_hbm.at[pl.ds(off, CK5)], ub)
            cx.wait()

            def edge_dot(k, _):
                z = jnp.zeros((LANES,), jnp.float32)
                a0, a1, a2 = z, z, z
                for j in range(C // LANES):
                    sl = pl.ds(j * LANES, LANES)
                    uv = ub[k, sl]
                    a0 = a0 + xsb[k, pl.ds(j * LANES, LANES)] * uv
                    a1 = a1 + xsb[k, pl.ds(C + j * LANES, LANES)] * uv
                    a2 = a2 + xsb[k, pl.ds(2 * C + j * LANES, LANES)] * uv
                cb[k, pl.ds(0, LANES)] = a0
                cb[k, pl.ds(LANES, LANES)] = a1
                cb[k, pl.ds(2 * LANES, LANES)] = a2
                return 0

            lax.fori_loop(0, CK5, edge_dot, 0)
            pltpu.sync_copy(cb, fshared.at[didx.at[0]], add=True)

        return 0

    lax.fori_loop(0, RR5, chunk, 0)
    plsc.subcore_barrier()

    @pl.when(sid == 0)
    def _():
        pltpu.sync_copy(fshared, fsrc_hbm.at[cid])


def _src_dots(zerosc, x134, u, src, dst):
    f = pl.kernel(
        _src_body,
        out_type=jax.ShapeDtypeStruct((NC, N_P, C), jnp.float32),
        mesh=plsc.VectorSubcoreMesh(**_SC_MESH),
        scratch_types=[
            pltpu.VMEM((CK5,), jnp.int32),
            pltpu.VMEM((1, CK5), jnp.int32),
            pltpu.VMEM((CK5, 3 * C), jnp.float32),
            pltpu.VMEM((CK5, C), jnp.float32),
            pltpu.VMEM((CK5, C), jnp.float32),
            pltpu.VMEM_SHARED((N_P, C), jnp.float32),
            pltpu.SemaphoreType.DMA,
        ],
    )
    return f(zerosc, x134, u, src, dst)


def _usum_body(zeros_hbm, u_hbm, dst_hbm, ssum_hbm, didx, ub, sshared):
    cid = lax.axis_index("c")
    sid = lax.axis_index("s")
    wid = sid * NC + cid

    @pl.when(sid == 0)
    def _():
        pltpu.sync_copy(zeros_hbm, sshared)

    plsc.subcore_barrier()

    def chunk(t, _):
        cidx = wid + NW * t

        @pl.when(cidx < TOT_CHUNKS)
        def _():
            off = cidx * CK
            pltpu.sync_copy(dst_hbm.at[pl.ds(off, CK)], didx.at[0])
            pltpu.sync_copy(u_hbm.at[pl.ds(off, CK)], ub)
            pltpu.sync_copy(ub, sshared.at[didx.at[0]], add=True)

        return 0

    lax.fori_loop(0, RR_ITERS, chunk, 0)
    plsc.subcore_barrier()

    @pl.when(sid == 0)
    def _():
        pltpu.sync_copy(sshared, ssum_hbm.at[cid])


def _u_scatter(zerosc, u, dst):
    f = pl.kernel(
        _usum_body,
        out_type=jax.ShapeDtypeStruct((NC, N_P, C), jnp.float32),
        mesh=plsc.VectorSubcoreMesh(**_SC_MESH),
        scratch_types=[
            pltpu.VMEM((1, CK), jnp.int32),
            pltpu.VMEM((CK, C), jnp.float32),
            pltpu.VMEM_SHARED((N_P, C), jnp.float32),
        ],
    )
    return f(zerosc, u, dst)


def _final_body(fsrc_ref, s_ref, x134_ref, out_ref):
    s = s_ref[0] + s_ref[1]
    f128 = fsrc_ref[0] + fsrc_ref[1]
    bn = f128.shape[0]
    iot = lax.broadcasted_iota(jnp.int32, (bn, 16), 1)
    acc = jnp.zeros((bn, 16), jnp.float32)
    for l in range(3):
        d = jnp.sum(x134_ref[...][:, l * C:(l + 1) * C] * s, axis=1,
                    keepdims=True)
        d = d + jnp.sum(f128[:, l * 16:(l + 1) * 16], axis=1, keepdims=True)
        acc = acc + jnp.where(iot == l, d, 0.0)
    out_ref[...] = acc


def _finalize(fsrc, ssum, x134):
    bn = 1024
    return pl.pallas_call(
        _final_body,
        grid=(N_P // bn,),
        in_specs=[
            pl.BlockSpec((NC, bn, C), lambda i: (0, i, 0)),
            pl.BlockSpec((NC, bn, C), lambda i: (0, i, 0)),
            pl.BlockSpec((bn, 3 * C), lambda i: (i, 0)),
        ],
        out_specs=pl.BlockSpec((bn, 16), lambda i: (i, 0)),
        out_shape=jax.ShapeDtypeStruct((N_P, 16), jnp.float32),
    )(fsrc, ssum, x134)


@jax.jit
def _run(x, atomic_numbers, edge_distance, edge_index, node_offset, p):
    src = edge_index[0].astype(jnp.int32)
    dst = (edge_index[1] - node_offset).astype(jnp.int32)

    an2 = jnp.full((N_P, 1), 127, jnp.int32).at[:N_N, 0].set(
        atomic_numbers.astype(jnp.int32))
    x0 = jnp.zeros((N_P, C), jnp.float32).at[:N_N].set(x[:, 0, :])
    x134 = jnp.zeros((N_P, 3 * C), jnp.float32).at[:N_N].set(
        x[:, 1:4, :].reshape(N_N, 3 * C))
    asrc_pad = jnp.zeros((128, C), jnp.float32).at[:90].set(p["atom_src"])
    adst_pad = jnp.zeros((128, C), jnp.float32).at[:90].set(p["atom_dst"])
    zerosc = jnp.zeros((N_P, C), jnp.float32)

    avflat = p["alpha_vec"].reshape(1, H * A)
    k_i = jnp.arange(H * A)[:, None] // A
    sel = (k_i == jnp.arange(C)[None, :]).astype(jnp.float32)
    rep = (jnp.arange(H)[:, None] == (jnp.arange(H * V)[None, :] // V)
           ).astype(jnp.float32)
    wvt = p["Wv"].T
    wo_row = p["Wo"][:, 0].reshape(1, H * V)
    wa1s = p["Wa1"][:C]
    wa1r = p["Wa1"][C:]

    u_tab, w_tab = _build_tables(an2, x0, asrc_pad, adst_pad)
    embcat = _edge_gather(u_tab, w_tab, src, dst)
    exw, gw = _edge_math(embcat, edge_distance, p["W_rad1"], p["W_rad2"],
                         wa1s, wa1r, avflat, sel, p["Wg"], wo_row)
    denp = _den_scatter(zerosc, exw, dst)
    den = _den_combine(denp)
    deng = _den_gather(den, dst)
    u = _u_project(exw, deng, gw, rep, wvt)
    fsrc = _src_dots(zerosc, x134, u, src, dst)
    ssum = _u_scatter(zerosc, u, dst)
    out16 = _finalize(fsrc, ssum, x134)
    return out16[:N_N, :3]


def kernel(x, atomic_numbers, edge_distance, edge_index, node_offset,
           params_force, params_denoise):
    del params_denoise
    return _run(x, atomic_numbers, edge_distance, edge_index,
                jnp.asarray(node_offset, jnp.int32), params_force)


# speedup vs baseline: 35.1309x; 1.0246x over previous
import jax
import jax.numpy as jnp
from jax import lax
from jax.experimental import pallas as pl
from jax.experimental.pallas import tpu as pltpu
from jax.experimental.pallas import tpu_sc as plsc

N_N = 10000
N_P = 10240
E_E = 160000
C = 128
H = 8
V = 16
A = 64
R = 64

NC = 2
NS = 16
NW = NC * NS
LANES = 16

CK = 128
TOT_CHUNKS = E_E // CK
RR_ITERS = (TOT_CHUNKS + NW - 1) // NW
CK5 = 64
TOT5 = E_E // CK5
RR5 = (TOT5 + NW - 1) // NW

_SC_MESH = dict(core_axis_name="c", subcore_axis_name="s")


def _tables_body(an_ref, x0_ref, asrc_ref, adst_ref, u_ref, w_ref):
    an = an_ref[...]
    iot = lax.broadcasted_iota(jnp.int32, (1, 128), 1)
    onehot = (an == iot).astype(jnp.float32)
    tsrc = jnp.dot(onehot, asrc_ref[...], preferred_element_type=jnp.float32)
    tdst = jnp.dot(onehot, adst_ref[...], preferred_element_type=jnp.float32)
    x0 = x0_ref[...]
    u_ref[...] = jnp.concatenate([tsrc, x0], axis=1)
    w_ref[...] = jnp.concatenate([tdst, x0], axis=1)


def _build_tables(an2, x0, asrc_pad, adst_pad):
    bn = 1024
    return pl.pallas_call(
        _tables_body,
        grid=(N_P // bn,),
        in_specs=[
            pl.BlockSpec((bn, 1), lambda i: (i, 0)),
            pl.BlockSpec((bn, C), lambda i: (i, 0)),
            pl.BlockSpec((128, C), lambda i: (0, 0)),
            pl.BlockSpec((128, C), lambda i: (0, 0)),
        ],
        out_specs=[
            pl.BlockSpec((bn, 2 * C), lambda i: (i, 0)),
            pl.BlockSpec((bn, 2 * C), lambda i: (i, 0)),
        ],
        out_shape=[
            jax.ShapeDtypeStruct((N_P, 2 * C), jnp.float32),
            jax.ShapeDtypeStruct((N_P, 2 * C), jnp.float32),
        ],
    )(an2, x0, asrc_pad, adst_pad)


def _edge_gather_body(u_hbm, w_hbm, src_hbm, dst_hbm, out_hbm,
                      sidx, didx, rows_u, rows_w, sem1, sem2):
    wid = lax.axis_index("s") * NC + lax.axis_index("c")

    def chunk(t, _):
        cidx = wid + NW * t

        @pl.when(cidx < TOT_CHUNKS)
        def _():
            off = cidx * CK
            pltpu.sync_copy(src_hbm.at[pl.ds(off, CK)], sidx)
            pltpu.sync_copy(dst_hbm.at[pl.ds(off, CK)], didx)
            cu = pltpu.async_copy(u_hbm.at[sidx], rows_u, sem1)
            cw = pltpu.async_copy(w_hbm.at[didx], rows_w, sem2)
            cu.wait()
            cw.wait()

            def addrow(k, _):
                for cc in range(2 * C // LANES):
                    sl = pl.ds(cc * LANES, LANES)
                    rows_u[k, sl] = rows_u[k, sl] + rows_w[k, sl]
                return 0

            lax.fori_loop(0, CK, addrow, 0)
            pltpu.sync_copy(rows_u, out_hbm.at[pl.ds(off, CK)])

        return 0

    lax.fori_loop(0, RR_ITERS, chunk, 0)


def _edge_gather(u_tab, w_tab, src, dst):
    f = pl.kernel(
        _edge_gather_body,
        out_type=jax.ShapeDtypeStruct((E_E, 2 * C), jnp.float32),
        mesh=plsc.VectorSubcoreMesh(**_SC_MESH),
        scratch_types=[
            pltpu.VMEM((CK,), jnp.int32),
            pltpu.VMEM((CK,), jnp.int32),
            pltpu.VMEM((CK, 2 * C), jnp.float32),
            pltpu.VMEM((CK, 2 * C), jnp.float32),
            pltpu.SemaphoreType.DMA,
            pltpu.SemaphoreType.DMA,
        ],
    )
    return f(u_tab, w_tab, src, dst)


def _edge_math_body(embcat_ref, ed_ref, wr1_ref, wr2_ref, wa1s_ref, wa1r_ref,
                    avflat_ref, sel_ref, wg_ref, wo_ref, exw_ref, gw_ref):
    embcat = embcat_ref[...]
    emb = embcat[:, :C] + ed_ref[...]
    scalar = embcat[:, C:]
    rad = jax.nn.silu(jnp.dot(emb, wr1_ref[...],
                              preferred_element_type=jnp.float32))
    rad = jax.nn.silu(jnp.dot(rad, wr2_ref[...],
                              preferred_element_type=jnp.float32))
    a = jax.nn.silu(
        jnp.dot(scalar, wa1s_ref[...], preferred_element_type=jnp.float32)
        + jnp.dot(rad, wa1r_ref[...], preferred_element_type=jnp.float32))
    aw = a * avflat_ref[...]
    logits128 = jnp.dot(aw, sel_ref[...], preferred_element_type=jnp.float32)
    exw_ref[...] = jnp.exp(logits128)
    gate = jax.nn.sigmoid(jnp.dot(rad, wg_ref[...],
                                  preferred_element_type=jnp.float32))
    gw_ref[...] = gate * wo_ref[...]


def _edge_math(embcat, edge_distance, wr1, wr2, wa1s, wa1r, avflat, sel, wg,
               wo_row):
    be = 2000
    return pl.pallas_call(
        _edge_math_body,
        grid=(E_E // be,),
        in_specs=[
            pl.BlockSpec((be, 2 * C), lambda i: (i, 0)),
            pl.BlockSpec((be, C), lambda i: (i, 0)),
            pl.BlockSpec((C, R), lambda i: (0, 0)),
            pl.BlockSpec((R, R), lambda i: (0, 0)),
            pl.BlockSpec((C, H * A), lambda i: (0, 0)),
            pl.BlockSpec((R, H * A), lambda i: (0, 0)),
            pl.BlockSpec((1, H * A), lambda i: (0, 0)),
            pl.BlockSpec((H * A, C), lambda i: (0, 0)),
            pl.BlockSpec((R, H * V), lambda i: (0, 0)),
            pl.BlockSpec((1, H * V), lambda i: (0, 0)),
        ],
        out_specs=[
            pl.BlockSpec((be, C), lambda i: (i, 0)),
            pl.BlockSpec((be, H * V), lambda i: (i, 0)),
        ],
        out_shape=[
            jax.ShapeDtypeStruct((E_E, C), jnp.float32),
            jax.ShapeDtypeStruct((E_E, H * V), jnp.float32),
        ],
    )(embcat, edge_distance, wr1, wr2, wa1s, wa1r, avflat, sel, wg, wo_row)


def _den_body(zeros_hbm, exw_hbm, dst_hbm, denp_hbm, idxb, rowsb, dshared,
              rsem):
    cid = lax.axis_index("c")
    sid = lax.axis_index("s")
    wid = sid * NC + cid

    @pl.when(sid == 0)
    def _():
        pltpu.sync_copy(zeros_hbm, dshared)

    plsc.subcore_barrier()

    def chunk(t, _):
        cidx = wid + NW * t

        @pl.when(cidx < TOT_CHUNKS)
        def _():
            off = cidx * CK
            cr = pltpu.async_copy(exw_hbm.at[pl.ds(off, CK)], rowsb, rsem)
            pltpu.sync_copy(dst_hbm.at[pl.ds(off, CK)], idxb.at[0])
            cr.wait()
            pltpu.sync_copy(rowsb, dshared.at[idxb.at[0]], add=True)

        return 0

    lax.fori_loop(0, RR_ITERS, chunk, 0)
    plsc.subcore_barrier()

    @pl.when(sid == 0)
    def _():
        pltpu.sync_copy(dshared, denp_hbm.at[cid])


def _den_scatter(zerosc, exw, dst):
    f = pl.kernel(
        _den_body,
        out_type=jax.ShapeDtypeStruct((NC, N_P, C), jnp.float32),
        mesh=plsc.VectorSubcoreMesh(**_SC_MESH),
        scratch_types=[
            pltpu.VMEM((1, CK), jnp.int32),
            pltpu.VMEM((CK, C), jnp.float32),
            pltpu.VMEM_SHARED((N_P, C), jnp.float32),
            pltpu.SemaphoreType.DMA,
        ],
    )
    return f(zerosc, exw, dst)


def _denc_body(denp_ref, den_ref):
    den_ref[...] = denp_ref[0] + denp_ref[1]


def _den_combine(denp):
    bn = 1024
    return pl.pallas_call(
        _denc_body,
        grid=(N_P // bn,),
        in_specs=[pl.BlockSpec((NC, bn, C), lambda i: (0, i, 0))],
        out_specs=pl.BlockSpec((bn, C), lambda i: (i, 0)),
        out_shape=jax.ShapeDtypeStruct((N_P, C), jnp.float32),
    )(denp)


def _deng_body(den_hbm, dst_hbm, deng_hbm, idxb, rows128, rowsb, sem):
    wid = lax.axis_index("s") * NC + lax.axis_index("c")

    def gchunk(t, _):
        cidx = wid + NW * t

        @pl.when(cidx < TOT_CHUNKS)
        def _():
            off = cidx * CK
            pltpu.sync_copy(dst_hbm.at[pl.ds(off, CK)], idxb)
            pltpu.async_copy(den_hbm.at[idxb], rows128, sem).wait()

            def narrow(k, _):
                rowsb[k, :] = rows128[k, pl.ds(0, LANES)]
                return 0

            lax.fori_loop(0, CK, narrow, 0)
            pltpu.sync_copy(rowsb, deng_hbm.at[pl.ds(off, CK)])

        return 0

    lax.fori_loop(0, RR_ITERS, gchunk, 0)


def _den_gather(den, dst):
    f = pl.kernel(
        _deng_body,
        out_type=jax.ShapeDtypeStruct((E_E, 16), jnp.float32),
        mesh=plsc.VectorSubcoreMesh(**_SC_MESH),
        scratch_types=[
            pltpu.VMEM((CK,), jnp.int32),
            pltpu.VMEM((CK, C), jnp.float32),
            pltpu.VMEM((CK, 16), jnp.float32),
            pltpu.SemaphoreType.DMA,
        ],
    )
    return f(den, dst)


def _u_body(exw_ref, deng_ref, gw_ref, rep_ref, wvt_ref, u_ref):
    ex = exw_ref[...][:, :H]
    den = deng_ref[...][:, :H]
    alpha = ex / (den + 1e-9)
    alpha_rep = jnp.dot(alpha, rep_ref[...], preferred_element_type=jnp.float32)
    u_ref[...] = jnp.dot(alpha_rep * gw_ref[...], wvt_ref[...],
                         preferred_element_type=jnp.float32)


def _u_project(exw, deng, gw, rep, wvt):
    be = 2000
    return pl.pallas_call(
        _u_body,
        grid=(E_E // be,),
        in_specs=[
            pl.BlockSpec((be, C), lambda i: (i, 0)),
            pl.BlockSpec((be, 16), lambda i: (i, 0)),
            pl.BlockSpec((be, H * V), lambda i: (i, 0)),
            pl.BlockSpec((H, H * V), lambda i: (0, 0)),
            pl.BlockSpec((H * V, C), lambda i: (0, 0)),
        ],
        out_specs=pl.BlockSpec((be, C), lambda i: (i, 0)),
        out_shape=jax.ShapeDtypeStruct((E_E, C), jnp.float32),
    )(exw, deng, gw, rep, wvt)


def _src_body(zeros_hbm, x134_hbm, u_hbm, src_hbm, dst_hbm, fsrc_hbm,
              sidx, didx, xsb, ub, cb, fshared, sem):
    cid = lax.axis_index("c")
    sid = lax.axis_index("s")
    wid = sid * NC + cid

    @pl.when(sid == 0)
    def _():
        pltpu.sync_copy(zeros_hbm, fshared)

    def zrow_c(j, _):
        for cc in range(C // LANES):
            cb[j, pl.ds(cc * LANES, LANES)] = jnp.zeros((LANES,), jnp.float32)
        return 0

    lax.fori_loop(0, CK5, zrow_c, 0)
    plsc.subcore_barrier()

    def chunk(t, _):
        cidx = wid + NW * t

        @pl.when(cidx < TOT5)
        def _():
            off = cidx * CK5
            pltpu.sync_copy(src_hbm.at[pl.ds(off, CK5)], sidx)
            pltpu.sync_copy(dst_hbm.at[pl.ds(off, CK5)], didx.at[0])
            cx = pltpu.async_copy(x134_hbm.at[sidx], xsb, sem)
            pltpu.sync_copy(u_hbm.at[pl.ds(off, CK5)], ub)
            cx.wait()

            def edge_dot(k, _):
                z = jnp.zeros((LANES,), jnp.float32)
                a0, a1, a2 = z, z, z
                for j in range(C // LANES):
                    sl = pl.ds(j * LANES, LANES)
                    uv = ub[k, sl]
                    a0 = a0 + xsb[k, pl.ds(j * LANES, LANES)] * uv
                    a1 = a1 + xsb[k, pl.ds(C + j * LANES, LANES)] * uv
                    a2 = a2 + xsb[k, pl.ds(2 * C + j * LANES, LANES)] * uv
                cb[k, pl.ds(0, LANES)] = a0
                cb[k, pl.ds(LANES, LANES)] = a1
                cb[k, pl.ds(2 * LANES, LANES)] = a2
                return 0

            lax.fori_loop(0, CK5, edge_dot, 0)
            pltpu.sync_copy(cb, fshared.at[didx.at[0]], add=True)

        return 0

    lax.fori_loop(0, RR5, chunk, 0)
    plsc.subcore_barrier()

    @pl.when(sid == 0)
    def _():
        pltpu.sync_copy(fshared, fsrc_hbm.at[cid])


def _src_dots(zerosc, x134, u, src, dst):
    f = pl.kernel(
        _src_body,
        out_type=jax.ShapeDtypeStruct((NC, N_P, C), jnp.float32),
        mesh=plsc.VectorSubcoreMesh(**_SC_MESH),
        scratch_types=[
            pltpu.VMEM((CK5,), jnp.int32),
            pltpu.VMEM((1, CK5), jnp.int32),
            pltpu.VMEM((CK5, 3 * C), jnp.float32),
            pltpu.VMEM((CK5, C), jnp.float32),
            pltpu.VMEM((CK5, C), jnp.float32),
            pltpu.VMEM_SHARED((N_P, C), jnp.float32),
            pltpu.SemaphoreType.DMA,
        ],
    )
    return f(zerosc, x134, u, src, dst)


def _usum_body(zeros_hbm, u_hbm, dst_hbm, ssum_hbm, didx, ub, sshared, rsem):
    cid = lax.axis_index("c")
    sid = lax.axis_index("s")
    wid = sid * NC + cid

    @pl.when(sid == 0)
    def _():
        pltpu.sync_copy(zeros_hbm, sshared)

    plsc.subcore_barrier()

    def chunk(t, _):
        cidx = wid + NW * t

        @pl.when(cidx < TOT_CHUNKS)
        def _():
            off = cidx * CK
            cr = pltpu.async_copy(u_hbm.at[pl.ds(off, CK)], ub, rsem)
            pltpu.sync_copy(dst_hbm.at[pl.ds(off, CK)], didx.at[0])
            cr.wait()
            pltpu.sync_copy(ub, sshared.at[didx.at[0]], add=True)

        return 0

    lax.fori_loop(0, RR_ITERS, chunk, 0)
    plsc.subcore_barrier()

    @pl.when(sid == 0)
    def _():
        pltpu.sync_copy(sshared, ssum_hbm.at[cid])


def _u_scatter(zerosc, u, dst):
    f = pl.kernel(
        _usum_body,
        out_type=jax.ShapeDtypeStruct((NC, N_P, C), jnp.float32),
        mesh=plsc.VectorSubcoreMesh(**_SC_MESH),
        scratch_types=[
            pltpu.VMEM((1, CK), jnp.int32),
            pltpu.VMEM((CK, C), jnp.float32),
            pltpu.VMEM_SHARED((N_P, C), jnp.float32),
            pltpu.SemaphoreType.DMA,
        ],
    )
    return f(zerosc, u, dst)


def _final_body(fsrc_ref, s_ref, x134_ref, out_ref):
    s = s_ref[0] + s_ref[1]
    f128 = fsrc_ref[0] + fsrc_ref[1]
    bn = f128.shape[0]
    iot = lax.broadcasted_iota(jnp.int32, (bn, 16), 1)
    acc = jnp.zeros((bn, 16), jnp.float32)
    for l in range(3):
        d = jnp.sum(x134_ref[...][:, l * C:(l + 1) * C] * s, axis=1,
                    keepdims=True)
        d = d + jnp.sum(f128[:, l * 16:(l + 1) * 16], axis=1, keepdims=True)
        acc = acc + jnp.where(iot == l, d, 0.0)
    out_ref[...] = acc


def _finalize(fsrc, ssum, x134):
    bn = 1024
    return pl.pallas_call(
        _final_body,
        grid=(N_P // bn,),
        in_specs=[
            pl.BlockSpec((NC, bn, C), lambda i: (0, i, 0)),
            pl.BlockSpec((NC, bn, C), lambda i: (0, i, 0)),
            pl.BlockSpec((bn, 3 * C), lambda i: (i, 0)),
        ],
        out_specs=pl.BlockSpec((bn, 16), lambda i: (i, 0)),
        out_shape=jax.ShapeDtypeStruct((N_P, 16), jnp.float32),
    )(fsrc, ssum, x134)


@jax.jit
def _run(x, atomic_numbers, edge_distance, edge_index, node_offset, p):
    src = edge_index[0].astype(jnp.int32)
    dst = (edge_index[1] - node_offset).astype(jnp.int32)

    an2 = jnp.full((N_P, 1), 127, jnp.int32).at[:N_N, 0].set(
        atomic_numbers.astype(jnp.int32))
    x0 = jnp.zeros((N_P, C), jnp.float32).at[:N_N].set(x[:, 0, :])
    x134 = jnp.zeros((N_P, 3 * C), jnp.float32).at[:N_N].set(
        x[:, 1:4, :].reshape(N_N, 3 * C))
    asrc_pad = jnp.zeros((128, C), jnp.float32).at[:90].set(p["atom_src"])
    adst_pad = jnp.zeros((128, C), jnp.float32).at[:90].set(p["atom_dst"])
    zerosc = jnp.zeros((N_P, C), jnp.float32)

    avflat = p["alpha_vec"].reshape(1, H * A)
    k_i = jnp.arange(H * A)[:, None] // A
    sel = (k_i == jnp.arange(C)[None, :]).astype(jnp.float32)
    rep = (jnp.arange(H)[:, None] == (jnp.arange(H * V)[None, :] // V)
           ).astype(jnp.float32)
    wvt = p["Wv"].T
    wo_row = p["Wo"][:, 0].reshape(1, H * V)
    wa1s = p["Wa1"][:C]
    wa1r = p["Wa1"][C:]

    u_tab, w_tab = _build_tables(an2, x0, asrc_pad, adst_pad)
    embcat = _edge_gather(u_tab, w_tab, src, dst)
    exw, gw = _edge_math(embcat, edge_distance, p["W_rad1"], p["W_rad2"],
                         wa1s, wa1r, avflat, sel, p["Wg"], wo_row)
    denp = _den_scatter(zerosc, exw, dst)
    den = _den_combine(denp)
    deng = _den_gather(den, dst)
    u = _u_project(exw, deng, gw, rep, wvt)
    fsrc = _src_dots(zerosc, x134, u, src, dst)
    ssum = _u_scatter(zerosc, u, dst)
    out16 = _finalize(fsrc, ssum, x134)
    return out16[:N_N, :3]


def kernel(x, atomic_numbers, edge_distance, edge_index, node_offset,
           params_force, params_denoise):
    del params_denoise
    return _run(x, atomic_numbers, edge_distance, edge_index,
                jnp.asarray(node_offset, jnp.int32), params_force)
